# baseline (speedup 1.0000x reference)
import functools

import jax
import jax.numpy as jnp
from jax import lax
from jax.experimental import pallas as pl
from jax.experimental.pallas import tpu as pltpu
from jax.experimental.pallas import tpu_sc as plsc

NN_K = 10
LOSS_EPS = 1e-8
SWEEPS = 10
_F32_EPS = float(jnp.finfo(jnp.float32).eps)


def _sc_gather(predT, idx12):
    RB, N = predT.shape
    B = idx12.shape[0]
    CH = 128
    chunks_per_row = N // CH
    n_chunks = RB * chunks_per_row
    n_workers = 32
    per_w = n_chunks // n_workers
    mesh = plsc.VectorSubcoreMesh(core_axis_name="c", subcore_axis_name="s")

    @functools.partial(
        pl.kernel,
        mesh=mesh,
        out_type=jax.ShapeDtypeStruct((RB, N), jnp.float32),
        scratch_types=[
            pltpu.VMEM((CH,), jnp.int32),
            pltpu.VMEM((CH,), jnp.float32),
            pltpu.SemaphoreType.DMA,
        ],
    )
    def k(pred_hbm, idx_hbm, out_hbm, idx_v, out_v, sem):
        wid = lax.axis_index("s") * 2 + lax.axis_index("c")
        for t in range(per_w):
            g = wid * per_w + t
            r = g // chunks_per_row
            off = (g % chunks_per_row) * CH
            b = lax.rem(r, B)
            pltpu.sync_copy(idx_hbm.at[b, pl.ds(off, CH)], idx_v)
            base = r * N

            def body(i, _):
                st = pl.multiple_of(i * 16, 16)
                idx_v[pl.ds(st, 16)] = idx_v[pl.ds(st, 16)] + base
                return 0

            lax.fori_loop(0, CH // 16, body, 0)
            pltpu.async_copy(pred_hbm.at[idx_v], out_v, sem).wait()
            pltpu.sync_copy(out_v, out_hbm.at[r, pl.ds(off, CH)])

    return k(predT.reshape(RB * N), idx12)


def _ka_body(q_ref, p_ref, s1_ref, s2_ref):
    P = p_ref[0]
    Q = q_ref[0]
    px, py, pz = P[0:1], P[1:2], P[2:3]
    sqp = px * px + py * py + pz * pz
    sqq = jnp.sum(Q * Q, axis=1, keepdims=True)
    dot = lax.dot_general(Q, P, dimension_numbers=(((1,), (0,)), ((), ())))
    d2 = (sqq + sqp) - 2.0 * dot
    iota = lax.broadcasted_iota(jnp.int32, d2.shape, 1)
    key = jnp.bitwise_or(
        jnp.bitwise_and(lax.bitcast_convert_type(d2 + 1.0, jnp.int32),
                        jnp.int32(-2048)), iota)
    keyf = lax.bitcast_convert_type(key, jnp.float32)
    inf = jnp.float32(jnp.inf)
    for _ in range(NN_K):
        mn = jnp.min(keyf, axis=1, keepdims=True)
        keyf = jnp.where(keyf == mn, inf, keyf)
    m = (keyf == inf).astype(jnp.float32)
    p9 = jnp.concatenate(
        [P, px * px, py * px, pz * px, py * py, pz * py, pz * pz],
        axis=0)
    mb = m.astype(jnp.bfloat16)
    p9h = p9.astype(jnp.bfloat16)
    r1 = p9 - p9h.astype(jnp.float32)
    p9m = r1.astype(jnp.bfloat16)
    p9l = (r1 - p9m.astype(jnp.float32)).astype(jnp.bfloat16)
    dims = (((1,), (1,)), ((), ()))
    f32 = jnp.float32
    s12 = (lax.dot_general(mb, p9h, dims, preferred_element_type=f32) +
           lax.dot_general(mb, p9m, dims, preferred_element_type=f32) +
           lax.dot_general(mb, p9l, dims, preferred_element_type=f32))
    s1_ref[0] = s12[:, 0:3]
    s2_ref[0] = s12[:, 3:9]


def _cov_sums(clouds_nt, clouds_t, bq, interpret=False):
    C, N, _ = clouds_nt.shape
    grid = (C, N // bq)
    return pl.pallas_call(
        _ka_body,
        grid=grid,
        in_specs=[
            pl.BlockSpec((1, bq, 3), lambda c, q: (c, q, 0)),
            pl.BlockSpec((1, 3, N), lambda c, q: (c, 0, 0)),
        ],
        out_specs=[
            pl.BlockSpec((1, bq, 3), lambda c, q: (c, q, 0)),
            pl.BlockSpec((1, bq, 6), lambda c, q: (c, q, 0)),
        ],
        out_shape=[
            jax.ShapeDtypeStruct((C, N, 3), jnp.float32),
            jax.ShapeDtypeStruct((C, N, 6), jnp.float32),
        ],
        interpret=interpret,
    )(clouds_nt, clouds_t)


def _jacobi_normal(c00, c01, c02, c11, c12, c22):
    W = [[c00, c01, c02], [c01, c11, c12], [c02, c12, c22]]
    one = jnp.ones_like(c00)
    zero = jnp.zeros_like(c00)
    V = [[one, zero, zero], [zero, one, zero], [zero, zero, one]]
    tiny = jnp.float32(0.1 * _F32_EPS)
    for _ in range(SWEEPS):
        for (p, q) in [(0, 2), (2, 1), (0, 1)]:
            app, aqq, apq = W[p][p], W[q][q], W[p][q]
            tau = (aqq - app) / (2.0 * apq)
            sq = jnp.sqrt(1.0 + tau * tau)
            t = 1.0 / (tau + jnp.where(tau >= 0, sq, -sq))
            off_tiny = jnp.abs(apq) <= tiny * jnp.minimum(
                jnp.abs(app), jnp.abs(aqq))
            t = jnp.where(off_tiny, 0.0, t)
            c = lax.rsqrt(1.0 + t * t)
            s = t * c
            rt1 = app - t * apq
            rt2 = aqq + t * apq
            for j in range(3):
                wp, wq = W[p][j], W[q][j]
                W[p][j] = wp * c - wq * s
                W[q][j] = wp * s + wq * c
            for i in range(3):
                wp, wq = W[i][p], W[i][q]
                W[i][p] = wp * c - wq * s
                W[i][q] = wp * s + wq * c
            W[p][p] = rt1
            W[q][q] = rt2
            W[p][q] = zero
            W[q][p] = zero
            for j in range(3):
                vp, vq = V[p][j], V[q][j]
                V[p][j] = vp * c - vq * s
                V[q][j] = vp * s + vq * c
    w0, w1, w2 = W[0][0], W[1][1], W[2][2]
    j1 = jnp.where(w1 < w0, 1, 0)
    wmin = jnp.where(w1 < w0, w1, w0)
    jstar = jnp.where(w2 < wmin, 2, j1)
    is0 = jstar == 0
    is1 = jstar == 1
    nx = jnp.where(is0, V[0][0], jnp.where(is1, V[1][0], V[2][0]))
    ny = jnp.where(is0, V[0][1], jnp.where(is1, V[1][1], V[2][1]))
    nz = jnp.where(is0, V[0][2], jnp.where(is1, V[1][2], V[2][2]))
    return nx, ny, nz


def _cov_from_sums(s1_ref, s2_ref):
    k = float(NN_K)
    mu0, mu1, mu2 = s1_ref[0] / k, s1_ref[1] / k, s1_ref[2] / k
    return (s2_ref[0] / k - mu0 * mu0, s2_ref[1] / k - mu1 * mu0,
            s2_ref[2] / k - mu2 * mu0, s2_ref[3] / k - mu1 * mu1,
            s2_ref[4] / k - mu2 * mu1, s2_ref[5] / k - mu2 * mu2)


def _kb_body(g1_ref, g2_ref, p1_ref, p2_ref, out_ref):
    gx, gy, gz = _jacobi_normal(*_cov_from_sums(g1_ref, g2_ref))
    ax, ay, az = _jacobi_normal(*_cov_from_sums(p1_ref, p2_ref))
    num = ax * gx + ay * gy + az * gz
    na = jnp.maximum(jnp.sqrt(ax * ax + ay * ay + az * az),
                     jnp.float32(LOSS_EPS))
    nb = jnp.maximum(jnp.sqrt(gx * gx + gy * gy + gz * gz),
                     jnp.float32(LOSS_EPS))
    out_ref[0] = jnp.reshape(jnp.sum(1.0 - num / (na * nb)), (1, 1))


def _loss_partials(g1, g2, p1, p2, interpret=False):
    G = 8
    sb = g1.shape[1] // G
    return pl.pallas_call(
        _kb_body,
        grid=(G,),
        in_specs=[
            pl.BlockSpec((3, sb, 128), lambda i: (0, i, 0)),
            pl.BlockSpec((6, sb, 128), lambda i: (0, i, 0)),
            pl.BlockSpec((3, sb, 128), lambda i: (0, i, 0)),
            pl.BlockSpec((6, sb, 128), lambda i: (0, i, 0)),
        ],
        out_specs=pl.BlockSpec((1, 1, 1), lambda i: (i, 0, 0)),
        out_shape=jax.ShapeDtypeStruct((G, 1, 1), jnp.float32),
        interpret=interpret,
    )(g1, g2, p1, p2)


def _loss_from_clouds(gt, pg, bq=512, interpret=False):
    B, N, _ = gt.shape
    gt_t = jnp.transpose(gt, (0, 2, 1))
    clouds_t = jnp.concatenate([gt_t, jnp.transpose(pg, (1, 0, 2))], axis=0)
    clouds_nt = jnp.transpose(clouds_t, (0, 2, 1))
    s1, s2 = _cov_sums(clouds_nt, clouds_t, bq, interpret)
    s1p = jnp.transpose(s1, (2, 0, 1)).reshape(3, 2 * B * N)
    s2p = jnp.transpose(s2, (2, 0, 1)).reshape(6, 2 * B * N)
    half = B * N
    rows = half // 128
    g1 = s1p[:, :half].reshape(3, rows, 128)
    p1 = s1p[:, half:].reshape(3, rows, 128)
    g2 = s2p[:, :half].reshape(6, rows, 128)
    p2 = s2p[:, half:].reshape(6, rows, 128)
    partials = _loss_partials(g1, g2, p1, p2, interpret)
    return jnp.sum(partials) / jnp.float32(B * N)


def kernel(gt, pred, idx12):
    B, N, D = gt.shape
    predT = jnp.transpose(pred, (2, 0, 1)).reshape(D * B, N)
    pg = _sc_gather(predT, idx12.astype(jnp.int32)).reshape(D, B, N)
    return _loss_from_clouds(gt, pg)


# speedup vs baseline: 1.0665x; 1.0665x over previous
import functools

import jax
import jax.numpy as jnp
from jax import lax
from jax.experimental import pallas as pl
from jax.experimental.pallas import tpu as pltpu
from jax.experimental.pallas import tpu_sc as plsc

NN_K = 10
LOSS_EPS = 1e-8
SWEEPS = 10
_F32_EPS = float(jnp.finfo(jnp.float32).eps)


def _sc_gather(predT, idx12):
    RB, N = predT.shape
    B = idx12.shape[0]
    CH = 128
    chunks_per_row = N // CH
    n_chunks = RB * chunks_per_row
    n_workers = 32
    per_w = n_chunks // n_workers
    mesh = plsc.VectorSubcoreMesh(core_axis_name="c", subcore_axis_name="s")

    @functools.partial(
        pl.kernel,
        mesh=mesh,
        out_type=jax.ShapeDtypeStruct((RB, N), jnp.float32),
        scratch_types=[
            pltpu.VMEM((CH,), jnp.int32),
            pltpu.VMEM((CH,), jnp.float32),
            pltpu.SemaphoreType.DMA,
        ],
    )
    def k(pred_hbm, idx_hbm, out_hbm, idx_v, out_v, sem):
        wid = lax.axis_index("s") * 2 + lax.axis_index("c")
        for t in range(per_w):
            g = wid * per_w + t
            r = g // chunks_per_row
            off = (g % chunks_per_row) * CH
            b = lax.rem(r, B)
            pltpu.sync_copy(idx_hbm.at[b, pl.ds(off, CH)], idx_v)
            base = r * N

            def body(i, _):
                st = pl.multiple_of(i * 16, 16)
                idx_v[pl.ds(st, 16)] = idx_v[pl.ds(st, 16)] + base
                return 0

            lax.fori_loop(0, CH // 16, body, 0)
            pltpu.async_copy(pred_hbm.at[idx_v], out_v, sem).wait()
            pltpu.sync_copy(out_v, out_hbm.at[r, pl.ds(off, CH)])

    return k(predT.reshape(RB * N), idx12)


def _ka_body(q_ref, p_ref, s1_ref, s2_ref):
    P = p_ref[0]
    Q = q_ref[0]
    px, py, pz = P[0:1], P[1:2], P[2:3]
    sqp = px * px + py * py + pz * pz
    sqq1 = jnp.sum(Q * Q, axis=1, keepdims=True) + 1.0
    dot = lax.dot_general(Q, P, dimension_numbers=(((1,), (0,)), ((), ())))
    d2 = (sqq1 + sqp) - 2.0 * dot
    iota = lax.broadcasted_iota(jnp.int32, (1, d2.shape[1]), 1)
    key = jnp.bitwise_or(
        jnp.bitwise_and(lax.bitcast_convert_type(d2, jnp.int32),
                        jnp.int32(-2048)), iota)
    keyf = lax.bitcast_convert_type(key, jnp.float32)
    inf = jnp.float32(jnp.inf)
    for _ in range(NN_K):
        mn = jnp.min(keyf, axis=1, keepdims=True)
        keyf = jnp.where(keyf == mn, inf, keyf)
    m = (keyf == inf).astype(jnp.float32)
    p9 = jnp.concatenate(
        [P, px * px, py * px, pz * px, py * py, pz * py, pz * pz],
        axis=0)
    mb = m.astype(jnp.bfloat16)
    p9h = p9.astype(jnp.bfloat16)
    p9l = (p9 - p9h.astype(jnp.float32)).astype(jnp.bfloat16)
    dims = (((1,), (1,)), ((), ()))
    f32 = jnp.float32
    s12 = (lax.dot_general(mb, p9h, dims, preferred_element_type=f32) +
           lax.dot_general(mb, p9l, dims, preferred_element_type=f32))
    s1_ref[0] = s12[:, 0:3]
    s2_ref[0] = s12[:, 3:9]


def _cov_sums(clouds_nt, clouds_t, bq, interpret=False):
    C, N, _ = clouds_nt.shape
    grid = (C, N // bq)
    return pl.pallas_call(
        _ka_body,
        grid=grid,
        in_specs=[
            pl.BlockSpec((1, bq, 3), lambda c, q: (c, q, 0)),
            pl.BlockSpec((1, 3, N), lambda c, q: (c, 0, 0)),
        ],
        out_specs=[
            pl.BlockSpec((1, bq, 3), lambda c, q: (c, q, 0)),
            pl.BlockSpec((1, bq, 6), lambda c, q: (c, q, 0)),
        ],
        out_shape=[
            jax.ShapeDtypeStruct((C, N, 3), jnp.float32),
            jax.ShapeDtypeStruct((C, N, 6), jnp.float32),
        ],
        interpret=interpret,
    )(clouds_nt, clouds_t)


def _jacobi_normal(c00, c01, c02, c11, c12, c22):
    W = [[c00, c01, c02], [c01, c11, c12], [c02, c12, c22]]
    one = jnp.ones_like(c00)
    zero = jnp.zeros_like(c00)
    V = [[one, zero, zero], [zero, one, zero], [zero, zero, one]]
    tiny = jnp.float32(0.1 * _F32_EPS)
    for _ in range(SWEEPS):
        for (p, q) in [(0, 2), (2, 1), (0, 1)]:
            app, aqq, apq = W[p][p], W[q][q], W[p][q]
            tau = (aqq - app) / (2.0 * apq)
            sq = jnp.sqrt(1.0 + tau * tau)
            t = 1.0 / (tau + jnp.where(tau >= 0, sq, -sq))
            off_tiny = jnp.abs(apq) <= tiny * jnp.minimum(
                jnp.abs(app), jnp.abs(aqq))
            t = jnp.where(off_tiny, 0.0, t)
            c = lax.rsqrt(1.0 + t * t)
            s = t * c
            rt1 = app - t * apq
            rt2 = aqq + t * apq
            for j in range(3):
                wp, wq = W[p][j], W[q][j]
                W[p][j] = wp * c - wq * s
                W[q][j] = wp * s + wq * c
            for i in range(3):
                wp, wq = W[i][p], W[i][q]
                W[i][p] = wp * c - wq * s
                W[i][q] = wp * s + wq * c
            W[p][p] = rt1
            W[q][q] = rt2
            W[p][q] = zero
            W[q][p] = zero
            for j in range(3):
                vp, vq = V[p][j], V[q][j]
                V[p][j] = vp * c - vq * s
                V[q][j] = vp * s + vq * c
    w0, w1, w2 = W[0][0], W[1][1], W[2][2]
    j1 = jnp.where(w1 < w0, 1, 0)
    wmin = jnp.where(w1 < w0, w1, w0)
    jstar = jnp.where(w2 < wmin, 2, j1)
    is0 = jstar == 0
    is1 = jstar == 1
    nx = jnp.where(is0, V[0][0], jnp.where(is1, V[1][0], V[2][0]))
    ny = jnp.where(is0, V[0][1], jnp.where(is1, V[1][1], V[2][1]))
    nz = jnp.where(is0, V[0][2], jnp.where(is1, V[1][2], V[2][2]))
    return nx, ny, nz


def _cov_from_sums(s1_ref, s2_ref):
    k = float(NN_K)
    mu0, mu1, mu2 = s1_ref[0] / k, s1_ref[1] / k, s1_ref[2] / k
    return (s2_ref[0] / k - mu0 * mu0, s2_ref[1] / k - mu1 * mu0,
            s2_ref[2] / k - mu2 * mu0, s2_ref[3] / k - mu1 * mu1,
            s2_ref[4] / k - mu2 * mu1, s2_ref[5] / k - mu2 * mu2)


def _kb_body(g1_ref, g2_ref, p1_ref, p2_ref, out_ref):
    gx, gy, gz = _jacobi_normal(*_cov_from_sums(g1_ref, g2_ref))
    ax, ay, az = _jacobi_normal(*_cov_from_sums(p1_ref, p2_ref))
    num = ax * gx + ay * gy + az * gz
    na = jnp.maximum(jnp.sqrt(ax * ax + ay * ay + az * az),
                     jnp.float32(LOSS_EPS))
    nb = jnp.maximum(jnp.sqrt(gx * gx + gy * gy + gz * gz),
                     jnp.float32(LOSS_EPS))
    out_ref[0] = jnp.reshape(jnp.sum(1.0 - num / (na * nb)), (1, 1))


def _loss_partials(g1, g2, p1, p2, interpret=False):
    G = 8
    sb = g1.shape[1] // G
    return pl.pallas_call(
        _kb_body,
        grid=(G,),
        in_specs=[
            pl.BlockSpec((3, sb, 128), lambda i: (0, i, 0)),
            pl.BlockSpec((6, sb, 128), lambda i: (0, i, 0)),
            pl.BlockSpec((3, sb, 128), lambda i: (0, i, 0)),
            pl.BlockSpec((6, sb, 128), lambda i: (0, i, 0)),
        ],
        out_specs=pl.BlockSpec((1, 1, 1), lambda i: (i, 0, 0)),
        out_shape=jax.ShapeDtypeStruct((G, 1, 1), jnp.float32),
        interpret=interpret,
    )(g1, g2, p1, p2)


def _loss_from_clouds(gt, pg, bq=512, interpret=False):
    B, N, _ = gt.shape
    gt_t = jnp.transpose(gt, (0, 2, 1))
    clouds_t = jnp.concatenate([gt_t, jnp.transpose(pg, (1, 0, 2))], axis=0)
    clouds_nt = jnp.transpose(clouds_t, (0, 2, 1))
    s1, s2 = _cov_sums(clouds_nt, clouds_t, bq, interpret)
    s1p = jnp.transpose(s1, (2, 0, 1)).reshape(3, 2 * B * N)
    s2p = jnp.transpose(s2, (2, 0, 1)).reshape(6, 2 * B * N)
    half = B * N
    rows = half // 128
    g1 = s1p[:, :half].reshape(3, rows, 128)
    p1 = s1p[:, half:].reshape(3, rows, 128)
    g2 = s2p[:, :half].reshape(6, rows, 128)
    p2 = s2p[:, half:].reshape(6, rows, 128)
    partials = _loss_partials(g1, g2, p1, p2, interpret)
    return jnp.sum(partials) / jnp.float32(B * N)


def kernel(gt, pred, idx12):
    B, N, D = gt.shape
    predT = jnp.transpose(pred, (2, 0, 1)).reshape(D * B, N)
    pg = _sc_gather(predT, idx12.astype(jnp.int32)).reshape(D, B, N)
    return _loss_from_clouds(gt, pg)


# speedup vs baseline: 1.0846x; 1.0170x over previous
import functools

import jax
import jax.numpy as jnp
from jax import lax
from jax.experimental import pallas as pl
from jax.experimental.pallas import tpu as pltpu
from jax.experimental.pallas import tpu_sc as plsc

NN_K = 10
LOSS_EPS = 1e-8
SWEEPS = 10
_F32_EPS = float(jnp.finfo(jnp.float32).eps)


def _sc_gather(predT, idx12):
    RB, N = predT.shape
    B = idx12.shape[0]
    CH = 128
    chunks_per_row = N // CH
    n_chunks = RB * chunks_per_row
    n_workers = 32
    per_w = n_chunks // n_workers
    mesh = plsc.VectorSubcoreMesh(core_axis_name="c", subcore_axis_name="s")

    @functools.partial(
        pl.kernel,
        mesh=mesh,
        out_type=jax.ShapeDtypeStruct((RB, N), jnp.float32),
        scratch_types=[
            pltpu.VMEM((CH,), jnp.int32),
            pltpu.VMEM((CH,), jnp.float32),
            pltpu.SemaphoreType.DMA,
        ],
    )
    def k(pred_hbm, idx_hbm, out_hbm, idx_v, out_v, sem):
        wid = lax.axis_index("s") * 2 + lax.axis_index("c")
        for t in range(per_w):
            g = wid * per_w + t
            r = g // chunks_per_row
            off = (g % chunks_per_row) * CH
            b = lax.rem(r, B)
            pltpu.sync_copy(idx_hbm.at[b, pl.ds(off, CH)], idx_v)
            base = r * N

            def body(i, _):
                st = pl.multiple_of(i * 16, 16)
                idx_v[pl.ds(st, 16)] = idx_v[pl.ds(st, 16)] + base
                return 0

            lax.fori_loop(0, CH // 16, body, 0)
            pltpu.async_copy(pred_hbm.at[idx_v], out_v, sem).wait()
            pltpu.sync_copy(out_v, out_hbm.at[r, pl.ds(off, CH)])

    return k(predT.reshape(RB * N), idx12)


def _ka_body(q_ref, p_ref, s1_ref, s2_ref):
    P = p_ref[0]
    Q = q_ref[0]
    px, py, pz = P[0:1], P[1:2], P[2:3]
    sqp = px * px + py * py + pz * pz
    sqq1 = jnp.sum(Q * Q, axis=1, keepdims=True) + 1.0
    dot = lax.dot_general(Q, P, dimension_numbers=(((1,), (0,)), ((), ())))
    d2 = (sqq1 + sqp) - 2.0 * dot
    iota = lax.broadcasted_iota(jnp.int32, (1, d2.shape[1]), 1)
    key = jnp.bitwise_or(
        jnp.bitwise_and(lax.bitcast_convert_type(d2, jnp.int32),
                        jnp.int32(-2048)), iota)
    keyf = lax.bitcast_convert_type(key, jnp.float32)
    inf = jnp.float32(jnp.inf)
    for _ in range(NN_K):
        mn = jnp.min(keyf, axis=1, keepdims=True)
        keyf = jnp.where(keyf == mn, inf, keyf)
    m = (keyf == inf).astype(jnp.float32)
    p9 = jnp.concatenate(
        [P, px * px, py * px, pz * px, py * py, pz * py, pz * pz],
        axis=0)
    mb = m.astype(jnp.bfloat16)
    p9h = p9.astype(jnp.bfloat16)
    p9l = (p9 - p9h.astype(jnp.float32)).astype(jnp.bfloat16)
    dims = (((1,), (1,)), ((), ()))
    f32 = jnp.float32
    s12 = (lax.dot_general(mb, p9h, dims, preferred_element_type=f32) +
           lax.dot_general(mb, p9l, dims, preferred_element_type=f32))
    s1_ref[0] = s12[:, 0:3]
    s2_ref[0] = s12[:, 3:9]


def _cov_sums(clouds_nt, clouds_t, bq, interpret=False):
    C, N, _ = clouds_nt.shape
    grid = (C, N // bq)
    return pl.pallas_call(
        _ka_body,
        grid=grid,
        in_specs=[
            pl.BlockSpec((1, bq, 3), lambda c, q: (c, q, 0)),
            pl.BlockSpec((1, 3, N), lambda c, q: (c, 0, 0)),
        ],
        out_specs=[
            pl.BlockSpec((1, bq, 3), lambda c, q: (c, q, 0)),
            pl.BlockSpec((1, bq, 6), lambda c, q: (c, q, 0)),
        ],
        out_shape=[
            jax.ShapeDtypeStruct((C, N, 3), jnp.float32),
            jax.ShapeDtypeStruct((C, N, 6), jnp.float32),
        ],
        interpret=interpret,
    )(clouds_nt, clouds_t)


def _jacobi_normal(c00, c01, c02, c11, c12, c22):
    W = [[c00, c01, c02], [c01, c11, c12], [c02, c12, c22]]
    one = jnp.ones_like(c00)
    zero = jnp.zeros_like(c00)
    V = [[one, zero, zero], [zero, one, zero], [zero, zero, one]]
    tiny = jnp.float32(0.1 * _F32_EPS)
    for _ in range(SWEEPS):
        for (p, q) in [(0, 2), (2, 1), (0, 1)]:
            app, aqq, apq = W[p][p], W[q][q], W[p][q]
            tau = (aqq - app) / (2.0 * apq)
            sq = jnp.sqrt(1.0 + tau * tau)
            t = 1.0 / (tau + jnp.where(tau >= 0, sq, -sq))
            off_tiny = jnp.abs(apq) <= tiny * jnp.minimum(
                jnp.abs(app), jnp.abs(aqq))
            t = jnp.where(off_tiny, 0.0, t)
            c = lax.rsqrt(1.0 + t * t)
            s = t * c
            rt1 = app - t * apq
            rt2 = aqq + t * apq
            for j in range(3):
                wp, wq = W[p][j], W[q][j]
                W[p][j] = wp * c - wq * s
                W[q][j] = wp * s + wq * c
            for i in range(3):
                wp, wq = W[i][p], W[i][q]
                W[i][p] = wp * c - wq * s
                W[i][q] = wp * s + wq * c
            W[p][p] = rt1
            W[q][q] = rt2
            W[p][q] = zero
            W[q][p] = zero
            for j in range(3):
                vp, vq = V[p][j], V[q][j]
                V[p][j] = vp * c - vq * s
                V[q][j] = vp * s + vq * c
    w0, w1, w2 = W[0][0], W[1][1], W[2][2]
    j1 = jnp.where(w1 < w0, 1, 0)
    wmin = jnp.where(w1 < w0, w1, w0)
    jstar = jnp.where(w2 < wmin, 2, j1)
    is0 = jstar == 0
    is1 = jstar == 1
    nx = jnp.where(is0, V[0][0], jnp.where(is1, V[1][0], V[2][0]))
    ny = jnp.where(is0, V[0][1], jnp.where(is1, V[1][1], V[2][1]))
    nz = jnp.where(is0, V[0][2], jnp.where(is1, V[1][2], V[2][2]))
    return nx, ny, nz


def _cov_from_sums(s1_ref, s2_ref):
    k = float(NN_K)
    mu0, mu1, mu2 = s1_ref[0] / k, s1_ref[1] / k, s1_ref[2] / k
    return (s2_ref[0] / k - mu0 * mu0, s2_ref[1] / k - mu1 * mu0,
            s2_ref[2] / k - mu2 * mu0, s2_ref[3] / k - mu1 * mu1,
            s2_ref[4] / k - mu2 * mu1, s2_ref[5] / k - mu2 * mu2)


def _kb_body(g1_ref, g2_ref, p1_ref, p2_ref, out_ref):
    gx, gy, gz = _jacobi_normal(*_cov_from_sums(g1_ref, g2_ref))
    ax, ay, az = _jacobi_normal(*_cov_from_sums(p1_ref, p2_ref))
    num = ax * gx + ay * gy + az * gz
    na = jnp.maximum(jnp.sqrt(ax * ax + ay * ay + az * az),
                     jnp.float32(LOSS_EPS))
    nb = jnp.maximum(jnp.sqrt(gx * gx + gy * gy + gz * gz),
                     jnp.float32(LOSS_EPS))
    out_ref[0] = jnp.reshape(jnp.sum(1.0 - num / (na * nb)), (1, 1))


def _loss_partials(g1, g2, p1, p2, interpret=False):
    G = 8
    sb = g1.shape[1] // G
    return pl.pallas_call(
        _kb_body,
        grid=(G,),
        in_specs=[
            pl.BlockSpec((3, sb, 128), lambda i: (0, i, 0)),
            pl.BlockSpec((6, sb, 128), lambda i: (0, i, 0)),
            pl.BlockSpec((3, sb, 128), lambda i: (0, i, 0)),
            pl.BlockSpec((6, sb, 128), lambda i: (0, i, 0)),
        ],
        out_specs=pl.BlockSpec((1, 1, 1), lambda i: (i, 0, 0)),
        out_shape=jax.ShapeDtypeStruct((G, 1, 1), jnp.float32),
        interpret=interpret,
    )(g1, g2, p1, p2)


def _loss_from_clouds(gt, pg, bq=1024, interpret=False):
    B, N, _ = gt.shape
    gt_t = jnp.transpose(gt, (0, 2, 1))
    clouds_t = jnp.concatenate([gt_t, jnp.transpose(pg, (1, 0, 2))], axis=0)
    clouds_nt = jnp.transpose(clouds_t, (0, 2, 1))
    s1, s2 = _cov_sums(clouds_nt, clouds_t, bq, interpret)
    s1p = jnp.transpose(s1, (2, 0, 1)).reshape(3, 2 * B * N)
    s2p = jnp.transpose(s2, (2, 0, 1)).reshape(6, 2 * B * N)
    half = B * N
    rows = half // 128
    g1 = s1p[:, :half].reshape(3, rows, 128)
    p1 = s1p[:, half:].reshape(3, rows, 128)
    g2 = s2p[:, :half].reshape(6, rows, 128)
    p2 = s2p[:, half:].reshape(6, rows, 128)
    partials = _loss_partials(g1, g2, p1, p2, interpret)
    return jnp.sum(partials) / jnp.float32(B * N)


def kernel(gt, pred, idx12):
    B, N, D = gt.shape
    predT = jnp.transpose(pred, (2, 0, 1)).reshape(D * B, N)
    pg = _sc_gather(predT, idx12.astype(jnp.int32)).reshape(D, B, N)
    return _loss_from_clouds(gt, pg)


# speedup vs baseline: 1.1814x; 1.0893x over previous
import functools

import jax
import jax.numpy as jnp
from jax import lax
from jax.experimental import pallas as pl
from jax.experimental.pallas import tpu as pltpu
from jax.experimental.pallas import tpu_sc as plsc

NN_K = 10
LOSS_EPS = 1e-8
SWEEPS = 10
_F32_EPS = float(jnp.finfo(jnp.float32).eps)


def _sc_gather(predT, idx12):
    RB, N = predT.shape
    B = idx12.shape[0]
    CH = 128
    chunks_per_row = N // CH
    n_chunks = RB * chunks_per_row
    n_workers = 32
    per_w = n_chunks // n_workers
    mesh = plsc.VectorSubcoreMesh(core_axis_name="c", subcore_axis_name="s")

    @functools.partial(
        pl.kernel,
        mesh=mesh,
        out_type=jax.ShapeDtypeStruct((RB, N), jnp.float32),
        scratch_types=[
            pltpu.VMEM((CH,), jnp.int32),
            pltpu.VMEM((CH,), jnp.float32),
            pltpu.SemaphoreType.DMA,
        ],
    )
    def k(pred_hbm, idx_hbm, out_hbm, idx_v, out_v, sem):
        wid = lax.axis_index("s") * 2 + lax.axis_index("c")
        for t in range(per_w):
            g = wid * per_w + t
            r = g // chunks_per_row
            off = (g % chunks_per_row) * CH
            b = lax.rem(r, B)
            pltpu.sync_copy(idx_hbm.at[b, pl.ds(off, CH)], idx_v)
            base = r * N

            def body(i, _):
                st = pl.multiple_of(i * 16, 16)
                idx_v[pl.ds(st, 16)] = idx_v[pl.ds(st, 16)] + base
                return 0

            lax.fori_loop(0, CH // 16, body, 0)
            pltpu.async_copy(pred_hbm.at[idx_v], out_v, sem).wait()
            pltpu.sync_copy(out_v, out_hbm.at[r, pl.ds(off, CH)])

    return k(predT.reshape(RB * N), idx12)


def _ka_body(p_ref, s12_ref):
    bq = s12_ref.shape[2]
    qi = pl.program_id(1)
    P = p_ref[0]
    Qt = p_ref[0, :, pl.ds(qi * bq, bq)]
    px, py, pz = P[0:1], P[1:2], P[2:3]
    sqp = px * px + py * py + pz * pz
    sqq1 = jnp.transpose(jnp.sum(Qt * Qt, axis=0, keepdims=True)) + 1.0
    dot = lax.dot_general(Qt, P, dimension_numbers=(((0,), (0,)), ((), ())))
    d2 = (sqq1 + sqp) - 2.0 * dot
    iota = lax.broadcasted_iota(jnp.int32, (1, d2.shape[1]), 1)
    key = jnp.bitwise_or(
        jnp.bitwise_and(lax.bitcast_convert_type(d2, jnp.int32),
                        jnp.int32(-2048)), iota)
    keyf = lax.bitcast_convert_type(key, jnp.float32)
    inf = jnp.float32(jnp.inf)
    for _ in range(NN_K):
        mn = jnp.min(keyf, axis=1, keepdims=True)
        keyf = jnp.where(keyf == mn, inf, keyf)
    m = (keyf == inf).astype(jnp.float32)
    p9 = jnp.concatenate(
        [P, px * px, py * px, pz * px, py * py, pz * py, pz * pz],
        axis=0)
    mb = m.astype(jnp.bfloat16)
    p9h = p9.astype(jnp.bfloat16)
    p9l = (p9 - p9h.astype(jnp.float32)).astype(jnp.bfloat16)
    dims = (((1,), (1,)), ((), ()))
    f32 = jnp.float32
    s12 = (lax.dot_general(mb, p9h, dims, preferred_element_type=f32) +
           lax.dot_general(mb, p9l, dims, preferred_element_type=f32))
    s12_ref[0] = jnp.transpose(s12)


def _cov_sums(clouds_t, bq, interpret=False):
    C, _, N = clouds_t.shape
    grid = (C, N // bq)
    return pl.pallas_call(
        _ka_body,
        grid=grid,
        in_specs=[
            pl.BlockSpec((1, 3, N), lambda c, q: (c, 0, 0)),
        ],
        out_specs=pl.BlockSpec((1, 9, bq), lambda c, q: (c, 0, q)),
        out_shape=jax.ShapeDtypeStruct((C, 9, N), jnp.float32),
        interpret=interpret,
    )(clouds_t)


def _jacobi_normal(c00, c01, c02, c11, c12, c22):
    W = [[c00, c01, c02], [c01, c11, c12], [c02, c12, c22]]
    one = jnp.ones_like(c00)
    zero = jnp.zeros_like(c00)
    V = [[one, zero, zero], [zero, one, zero], [zero, zero, one]]
    tiny = jnp.float32(0.1 * _F32_EPS)
    for _ in range(SWEEPS):
        for (p, q) in [(0, 2), (2, 1), (0, 1)]:
            app, aqq, apq = W[p][p], W[q][q], W[p][q]
            tau = (aqq - app) / (2.0 * apq)
            sq = jnp.sqrt(1.0 + tau * tau)
            t = 1.0 / (tau + jnp.where(tau >= 0, sq, -sq))
            off_tiny = jnp.abs(apq) <= tiny * jnp.minimum(
                jnp.abs(app), jnp.abs(aqq))
            t = jnp.where(off_tiny, 0.0, t)
            c = lax.rsqrt(1.0 + t * t)
            s = t * c
            rt1 = app - t * apq
            rt2 = aqq + t * apq
            for j in range(3):
                wp, wq = W[p][j], W[q][j]
                W[p][j] = wp * c - wq * s
                W[q][j] = wp * s + wq * c
            for i in range(3):
                wp, wq = W[i][p], W[i][q]
                W[i][p] = wp * c - wq * s
                W[i][q] = wp * s + wq * c
            W[p][p] = rt1
            W[q][q] = rt2
            W[p][q] = zero
            W[q][p] = zero
            for j in range(3):
                vp, vq = V[p][j], V[q][j]
                V[p][j] = vp * c - vq * s
                V[q][j] = vp * s + vq * c
    w0, w1, w2 = W[0][0], W[1][1], W[2][2]
    j1 = jnp.where(w1 < w0, 1, 0)
    wmin = jnp.where(w1 < w0, w1, w0)
    jstar = jnp.where(w2 < wmin, 2, j1)
    is0 = jstar == 0
    is1 = jstar == 1
    nx = jnp.where(is0, V[0][0], jnp.where(is1, V[1][0], V[2][0]))
    ny = jnp.where(is0, V[0][1], jnp.where(is1, V[1][1], V[2][1]))
    nz = jnp.where(is0, V[0][2], jnp.where(is1, V[1][2], V[2][2]))
    return nx, ny, nz


def _cov_from_sums(s1_ref, s2_ref):
    k = float(NN_K)
    mu0, mu1, mu2 = s1_ref[0] / k, s1_ref[1] / k, s1_ref[2] / k
    return (s2_ref[0] / k - mu0 * mu0, s2_ref[1] / k - mu1 * mu0,
            s2_ref[2] / k - mu2 * mu0, s2_ref[3] / k - mu1 * mu1,
            s2_ref[4] / k - mu2 * mu1, s2_ref[5] / k - mu2 * mu2)


def _kb_body(g1_ref, g2_ref, p1_ref, p2_ref, out_ref):
    gx, gy, gz = _jacobi_normal(*_cov_from_sums(g1_ref, g2_ref))
    ax, ay, az = _jacobi_normal(*_cov_from_sums(p1_ref, p2_ref))
    num = ax * gx + ay * gy + az * gz
    na = jnp.maximum(jnp.sqrt(ax * ax + ay * ay + az * az),
                     jnp.float32(LOSS_EPS))
    nb = jnp.maximum(jnp.sqrt(gx * gx + gy * gy + gz * gz),
                     jnp.float32(LOSS_EPS))
    out_ref[0] = jnp.reshape(jnp.sum(1.0 - num / (na * nb)), (1, 1))


def _loss_partials(g1, g2, p1, p2, interpret=False):
    G = 8
    sb = g1.shape[1] // G
    return pl.pallas_call(
        _kb_body,
        grid=(G,),
        in_specs=[
            pl.BlockSpec((3, sb, 128), lambda i: (0, i, 0)),
            pl.BlockSpec((6, sb, 128), lambda i: (0, i, 0)),
            pl.BlockSpec((3, sb, 128), lambda i: (0, i, 0)),
            pl.BlockSpec((6, sb, 128), lambda i: (0, i, 0)),
        ],
        out_specs=pl.BlockSpec((1, 1, 1), lambda i: (i, 0, 0)),
        out_shape=jax.ShapeDtypeStruct((G, 1, 1), jnp.float32),
        interpret=interpret,
    )(g1, g2, p1, p2)


def _loss_from_clouds(gt, pg, bq=1024, interpret=False):
    B, N, _ = gt.shape
    gt_t = jnp.transpose(gt, (0, 2, 1))
    clouds_t = jnp.concatenate([gt_t, jnp.transpose(pg, (1, 0, 2))], axis=0)
    s12 = _cov_sums(clouds_t, bq, interpret)
    planes = jnp.transpose(s12, (1, 0, 2)).reshape(9, 2 * B * N)
    half = B * N
    rows = half // 128
    g1 = planes[0:3, :half].reshape(3, rows, 128)
    p1 = planes[0:3, half:].reshape(3, rows, 128)
    g2 = planes[3:9, :half].reshape(6, rows, 128)
    p2 = planes[3:9, half:].reshape(6, rows, 128)
    partials = _loss_partials(g1, g2, p1, p2, interpret)
    return jnp.sum(partials) / jnp.float32(B * N)


def kernel(gt, pred, idx12):
    B, N, D = gt.shape
    predT = jnp.transpose(pred, (2, 0, 1)).reshape(D * B, N)
    pg = _sc_gather(predT, idx12.astype(jnp.int32)).reshape(D, B, N)
    return _loss_from_clouds(gt, pg)
